# speedup vs baseline: 48.9509x; 48.9509x over previous
import functools
import math

import jax
import jax.numpy as jnp
from jax import lax
from jax.experimental import pallas as pl
from jax.experimental.pallas import tpu as pltpu

_INTERPRET = False

H = 4
C = 32


def _matmul_kernel(x_ref, w_ref, o_ref):
    o_ref[...] = jnp.dot(x_ref[...], w_ref[...],
                         preferred_element_type=jnp.float32)


def _matmul(x, w, bm=512):
    n, k = x.shape
    m = w.shape[1]
    return pl.pallas_call(
        _matmul_kernel,
        grid=(n // bm,),
        in_specs=[pl.BlockSpec((bm, k), lambda i: (i, 0)),
                  pl.BlockSpec((k, m), lambda i: (0, 0))],
        out_specs=pl.BlockSpec((bm, m), lambda i: (i, 0)),
        out_shape=jax.ShapeDtypeStruct((n, m), jnp.float32),
        interpret=_INTERPRET,
    )(x, w)


def _gat_kernel(cnt_ref, xp_ref, asrc_ref, bdst_ref, xres_ref, gb_ref,
                lng_ref, lnb_ref, o_ref, acc_ref, den_ref, *, bd, bs, nsb):
    i = pl.program_id(0)
    j = pl.program_id(1)

    @pl.when(j == 0)
    def _():
        acc_ref[...] = jnp.zeros_like(acc_ref)
        den_ref[...] = jnp.zeros_like(den_ref)

    cnt = cnt_ref[...]
    rows = i * bd + lax.broadcasted_iota(jnp.int32, (bd, bs), 0)
    cols = j * bs + lax.broadcasted_iota(jnp.int32, (bd, bs), 1)
    cnt = cnt + jnp.where(rows == cols, 1.0, 0.0)

    for hh in range(H):
        a1 = asrc_ref[hh, :]
        a2 = asrc_ref[H + hh, :]
        b1 = bdst_ref[hh, :]
        b2 = bdst_ref[H + hh, :]
        m = jnp.maximum(b1[:, None] * a1[None, :], b2[:, None] * a2[None, :])
        w = m * cnt
        acc_ref[:, hh * C:(hh + 1) * C] += jnp.dot(
            w, xp_ref[:, hh * C:(hh + 1) * C],
            preferred_element_type=jnp.float32)
        den_ref[hh, :] += jnp.sum(w, axis=1)

    @pl.when(j == nsb - 1)
    def _():
        y = acc_ref[...]
        den = den_ref[...]
        parts = [y[:, hh * C:(hh + 1) * C] / den[hh, :][:, None]
                 for hh in range(H)]
        y = jnp.concatenate(parts, axis=1) + gb_ref[...]
        mu = jnp.mean(y, axis=1, keepdims=True)
        var = jnp.mean((y - mu) ** 2, axis=1, keepdims=True)
        y = (y - mu) * lax.rsqrt(var + 1e-5) * lng_ref[...] + lnb_ref[...]
        o_ref[...] = y + xres_ref[...]


def _gat_layer(cnt, xp, asrc, bdst, xres, gbias, lng, lnb, bd=256, bs=512):
    n, d = xp.shape
    gd, gs = n // bd, n // bs
    kern = functools.partial(_gat_kernel, bd=bd, bs=bs, nsb=gs)
    return pl.pallas_call(
        kern,
        grid=(gd, gs),
        in_specs=[
            pl.BlockSpec((bd, bs), lambda i, j: (i, j)),
            pl.BlockSpec((bs, d), lambda i, j: (j, 0)),
            pl.BlockSpec((2 * H, bs), lambda i, j: (0, j)),
            pl.BlockSpec((2 * H, bd), lambda i, j: (0, i)),
            pl.BlockSpec((bd, d), lambda i, j: (i, 0)),
            pl.BlockSpec((1, d), lambda i, j: (0, 0)),
            pl.BlockSpec((1, d), lambda i, j: (0, 0)),
            pl.BlockSpec((1, d), lambda i, j: (0, 0)),
        ],
        out_specs=pl.BlockSpec((bd, d), lambda i, j: (i, 0)),
        out_shape=jax.ShapeDtypeStruct((n, d), jnp.float32),
        scratch_shapes=[pltpu.VMEM((bd, d), jnp.float32),
                        pltpu.VMEM((2 * H, bd), jnp.float32)],
        interpret=_INTERPRET,
    )(cnt, xp, asrc, bdst, xres, gbias, lng, lnb)


def _mlp_kernel(x_ref, ew1_ref, eb1_ref, ew2_ref, eb2_ref, dw1_ref, db1_ref,
                dw2_ref, db2_ref, iw_ref, ib_ref, o_ref):
    x = x_ref[...]
    h1 = jnp.maximum(jnp.dot(x, ew1_ref[...],
                             preferred_element_type=jnp.float32)
                     + eb1_ref[...], 0.0)
    msg = jnp.dot(h1, ew2_ref[...],
                  preferred_element_type=jnp.float32) + eb2_ref[...]
    d1 = jnp.maximum(jnp.dot(msg, dw1_ref[...],
                             preferred_element_type=jnp.float32)
                     + db1_ref[...], 0.0)
    dec = jnp.dot(d1, dw2_ref[...],
                  preferred_element_type=jnp.float32) + db2_ref[...]
    o_ref[...] = jnp.dot(dec, iw_ref[...],
                         preferred_element_type=jnp.float32) + ib_ref[...]


def _mlp_qkv(x, ew1, eb1, ew2, eb2, dw1, db1, dw2, db2, iw, ib, bm=512):
    n, d = x.shape
    m = iw.shape[1]
    full = lambda a: pl.BlockSpec(a.shape, lambda i: tuple(0 for _ in a.shape))
    return pl.pallas_call(
        _mlp_kernel,
        grid=(n // bm,),
        in_specs=[pl.BlockSpec((bm, d), lambda i: (i, 0)),
                  full(ew1), full(eb1), full(ew2), full(eb2),
                  full(dw1), full(db1), full(dw2), full(db2),
                  full(iw), full(ib)],
        out_specs=pl.BlockSpec((bm, m), lambda i: (i, 0)),
        out_shape=jax.ShapeDtypeStruct((n, m), jnp.float32),
        interpret=_INTERPRET,
    )(x, ew1, eb1, ew2, eb2, dw1, db1, dw2, db2, iw, ib)


def _mha_kernel(qkv_ref, qblk_ref, st_ref, ow_ref, ob_ref, gw1_ref, gb1_ref,
                g2_ref, b2_ref, pw_ref, pb_ref, o_ref, *, bq, bk, n):
    scale = 1.0 / math.sqrt(C)
    outs = []
    for hh in range(H):
        q = qblk_ref[:, hh * C:(hh + 1) * C]

        def body(kb, carry, q=q, hh=hh):
            m, l, acc = carry
            k = qkv_ref[pl.ds(kb * bk, bk), 128 + hh * C:128 + (hh + 1) * C]
            v = qkv_ref[pl.ds(kb * bk, bk), 256 + hh * C:256 + (hh + 1) * C]
            s = lax.dot_general(q, k, (((1,), (1,)), ((), ())),
                                preferred_element_type=jnp.float32) * scale
            mn = jnp.maximum(m, jnp.max(s, axis=1, keepdims=True))
            p = jnp.exp(s - mn)
            sc = jnp.exp(m - mn)
            l2 = l * sc + jnp.sum(p, axis=1, keepdims=True)
            acc2 = acc * sc + jnp.dot(p, v, preferred_element_type=jnp.float32)
            return mn, l2, acc2

        m0 = jnp.full((bq, 1), -1e30, jnp.float32)
        l0 = jnp.zeros((bq, 1), jnp.float32)
        a0 = jnp.zeros((bq, C), jnp.float32)
        m, l, acc = lax.fori_loop(0, n // bk, body, (m0, l0, a0))
        outs.append(acc / l)
    o = jnp.concatenate(outs, axis=1)
    agg = jnp.dot(o, ow_ref[...], preferred_element_type=jnp.float32) \
        + ob_ref[...]
    st = st_ref[...]
    gi = jnp.concatenate([st, agg], axis=1)
    hg = jnp.maximum(jnp.dot(gi, gw1_ref[...],
                             preferred_element_type=jnp.float32)
                     + gb1_ref[...], 0.0)
    s1 = jnp.sum(hg * g2_ref[...], axis=1)[:, None]
    strength = jax.nn.sigmoid(s1 + b2_ref[...])
    gated = agg * strength
    o_ref[...] = jnp.dot(gated, pw_ref[...],
                         preferred_element_type=jnp.float32) \
        + pb_ref[...] + st


def _mha_gate(qkv, states, ow, ob, gw1, gb1, g2, b2, pw, pb, bq=256, bk=512):
    n = qkv.shape[0]
    d = states.shape[1]
    full = lambda a: pl.BlockSpec(a.shape, lambda i: tuple(0 for _ in a.shape))
    kern = functools.partial(_mha_kernel, bq=bq, bk=bk, n=n)
    return pl.pallas_call(
        kern,
        grid=(n // bq,),
        in_specs=[full(qkv),
                  pl.BlockSpec((bq, qkv.shape[1]), lambda i: (i, 0)),
                  pl.BlockSpec((bq, d), lambda i: (i, 0)),
                  full(ow), full(ob), full(gw1), full(gb1),
                  full(g2), full(b2), full(pw), full(pb)],
        out_specs=pl.BlockSpec((bq, d), lambda i: (i, 0)),
        out_shape=jax.ShapeDtypeStruct((n, d), jnp.float32),
        interpret=_INTERPRET,
    )(qkv, qkv, states, ow, ob, gw1, gb1, g2, b2, pw, pb)


def _build_cnt(edge_index, n):
    src = edge_index[0]
    dst = edge_index[1]
    return jnp.zeros((n, n), jnp.float32).at[dst, src].add(1.0)


def _head_blockdiag(att):
    cols = [jnp.zeros((H * C,), jnp.float32).at[h * C:(h + 1) * C]
            .set(att[h]) for h in range(H)]
    return jnp.stack(cols, axis=1)


def kernel(agent_states, edge_index, params):
    p = params
    n, d = agent_states.shape

    cnt = _build_cnt(edge_index, n)

    x = agent_states + p['agent_emb'] + jnp.tile(p['role_emb'], (1, 4))
    for l in range(2):
        gp = p['gat'][l]
        s_m = _head_blockdiag(gp['att_src'])
        d_m = _head_blockdiag(gp['att_dst'])
        wc = jnp.concatenate([gp['W'], gp['W'] @ s_m, gp['W'] @ d_m], axis=1)
        t = _matmul(x, wc)
        xp = t[:, :d]
        asr = t[:, d:d + H]
        adr = t[:, d + H:d + 2 * H]
        ms = jnp.max(asr, axis=0)
        zs = adr + ms[None, :]
        b = jnp.maximum(zs, 0.2 * zs)
        a1 = jnp.exp(asr - ms[None, :])
        a2 = jnp.exp(0.2 * (asr - ms[None, :]))
        b1 = jnp.exp(zs - b)
        b2 = jnp.exp(0.2 * zs - b)
        asrc = jnp.concatenate([a1.T, a2.T], axis=0)
        bdst = jnp.concatenate([b1.T, b2.T], axis=0)
        x = _gat_layer(cnt, xp, asrc, bdst, x, gp['bias'][None, :],
                       p['ln_g'][l][None, :], p['ln_b'][l][None, :])

    qkv = _mlp_qkv(x, p['enc_W1'], p['enc_b1'][None, :],
                   p['enc_W2'], p['enc_b2'][None, :],
                   p['dec_W1'], p['dec_b1'][None, :],
                   p['dec_W2'], p['dec_b2'][None, :],
                   p['mha_in_W'], p['mha_in_b'][None, :])

    g2 = p['gate_W2'][:, 0][None, :]
    b2 = jnp.broadcast_to(p['gate_b2'], (1, d))
    out = _mha_gate(qkv, agent_states,
                    p['mha_out_W'], p['mha_out_b'][None, :],
                    p['gate_W1'], p['gate_b1'][None, :],
                    g2, b2,
                    p['proj_W'], p['proj_b'][None, :])
    return out


# speedup vs baseline: 71.9901x; 1.4707x over previous
import functools
import math

import jax
import jax.numpy as jnp
from jax import lax
from jax.experimental import pallas as pl
from jax.experimental.pallas import tpu as pltpu
from jax.experimental.pallas import tpu_sc as plsc

_INTERPRET = False

H = 4
C = 32


def _matmul_kernel(x_ref, w_ref, o_ref):
    o_ref[...] = jnp.dot(x_ref[...], w_ref[...],
                         preferred_element_type=jnp.float32)


def _matmul(x, w, bm=512):
    n, k = x.shape
    m = w.shape[1]
    return pl.pallas_call(
        _matmul_kernel,
        grid=(n // bm,),
        in_specs=[pl.BlockSpec((bm, k), lambda i: (i, 0)),
                  pl.BlockSpec((k, m), lambda i: (0, 0))],
        out_specs=pl.BlockSpec((bm, m), lambda i: (i, 0)),
        out_shape=jax.ShapeDtypeStruct((n, m), jnp.float32),
        interpret=_INTERPRET,
    )(x, w)


def _gat_kernel(cnt_ref, va_ref, asrc_ref, bdst_ref, xres_ref, gb_ref,
                lng_ref, lnb_ref, o_ref, acc_ref, *, bd, bs, nsb):
    j = pl.program_id(1)

    @pl.when(j == 0)
    def _():
        acc_ref[...] = jnp.zeros_like(acc_ref)

    cnt = cnt_ref[...]
    for hh in range(H):
        a1 = asrc_ref[hh, :][None, :]
        a2 = asrc_ref[H + hh, :][None, :]
        b1 = bdst_ref[:, hh:hh + 1]
        b2 = bdst_ref[:, H + hh:H + hh + 1]
        w = jnp.maximum(b1 * a1, b2 * a2) * cnt
        acc_ref[:, 40 * hh:40 * hh + 40] += jnp.dot(
            w, va_ref[:, 40 * hh:40 * hh + 40],
            preferred_element_type=jnp.float32)

    @pl.when(j == nsb - 1)
    def _():
        y = acc_ref[...]
        parts = [y[:, 40 * hh:40 * hh + C] / y[:, 40 * hh + C:40 * hh + C + 1]
                 for hh in range(H)]
        y = jnp.concatenate(parts, axis=1) + gb_ref[...]
        mu = jnp.mean(y, axis=1, keepdims=True)
        var = jnp.mean((y - mu) ** 2, axis=1, keepdims=True)
        y = (y - mu) * lax.rsqrt(var + 1e-5) * lng_ref[...] + lnb_ref[...]
        o_ref[...] = y + xres_ref[...]


def _gat_layer(cnt, va, asrc, bdst, xres, gbias, lng, lnb, bd=256, bs=512):
    n, d = xres.shape
    gd, gs = n // bd, n // bs
    kern = functools.partial(_gat_kernel, bd=bd, bs=bs, nsb=gs)
    return pl.pallas_call(
        kern,
        grid=(gd, gs),
        in_specs=[
            pl.BlockSpec((bd, bs), lambda i, j: (i, j)),
            pl.BlockSpec((bs, 40 * H), lambda i, j: (j, 0)),
            pl.BlockSpec((2 * H, bs), lambda i, j: (0, j)),
            pl.BlockSpec((bd, 2 * H), lambda i, j: (i, 0)),
            pl.BlockSpec((bd, d), lambda i, j: (i, 0)),
            pl.BlockSpec((1, d), lambda i, j: (0, 0)),
            pl.BlockSpec((1, d), lambda i, j: (0, 0)),
            pl.BlockSpec((1, d), lambda i, j: (0, 0)),
        ],
        out_specs=pl.BlockSpec((bd, d), lambda i, j: (i, 0)),
        out_shape=jax.ShapeDtypeStruct((n, d), jnp.float32),
        scratch_shapes=[pltpu.VMEM((bd, 40 * H), jnp.float32)],
        interpret=_INTERPRET,
    )(cnt, va, asrc, bdst, xres, gbias, lng, lnb)


def _mlp_kernel(x_ref, ew1_ref, eb1_ref, ew2_ref, eb2_ref, dw1_ref, db1_ref,
                dw2_ref, db2_ref, iw_ref, ib_ref, o_ref):
    x = x_ref[...]
    h1 = jnp.maximum(jnp.dot(x, ew1_ref[...],
                             preferred_element_type=jnp.float32)
                     + eb1_ref[...], 0.0)
    msg = jnp.dot(h1, ew2_ref[...],
                  preferred_element_type=jnp.float32) + eb2_ref[...]
    d1 = jnp.maximum(jnp.dot(msg, dw1_ref[...],
                             preferred_element_type=jnp.float32)
                     + db1_ref[...], 0.0)
    dec = jnp.dot(d1, dw2_ref[...],
                  preferred_element_type=jnp.float32) + db2_ref[...]
    o_ref[...] = jnp.dot(dec, iw_ref[...],
                         preferred_element_type=jnp.float32) + ib_ref[...]


def _mlp_qkv(x, ew1, eb1, ew2, eb2, dw1, db1, dw2, db2, iw, ib, bm=512):
    n, d = x.shape
    m = iw.shape[1]
    full = lambda a: pl.BlockSpec(a.shape, lambda i: tuple(0 for _ in a.shape))
    return pl.pallas_call(
        _mlp_kernel,
        grid=(n // bm,),
        in_specs=[pl.BlockSpec((bm, d), lambda i: (i, 0)),
                  full(ew1), full(eb1), full(ew2), full(eb2),
                  full(dw1), full(db1), full(dw2), full(db2),
                  full(iw), full(ib)],
        out_specs=pl.BlockSpec((bm, m), lambda i: (i, 0)),
        out_shape=jax.ShapeDtypeStruct((n, m), jnp.float32),
        interpret=_INTERPRET,
    )(x, ew1, eb1, ew2, eb2, dw1, db1, dw2, db2, iw, ib)


def _mha_kernel(qkv_ref, qblk_ref, st_ref, ow_ref, ob_ref, gw1_ref, gb1_ref,
                g2_ref, b2_ref, pw_ref, pb_ref, o_ref, *, bq, bk, n):
    scale = 1.0 / math.sqrt(C)
    outs = []
    for hh in range(H):
        q = qblk_ref[:, hh * C:(hh + 1) * C]

        def body(kb, carry, q=q, hh=hh):
            m, l, acc = carry
            k = qkv_ref[pl.ds(kb * bk, bk), 128 + hh * C:128 + (hh + 1) * C]
            v = qkv_ref[pl.ds(kb * bk, bk), 256 + hh * C:256 + (hh + 1) * C]
            s = lax.dot_general(q, k, (((1,), (1,)), ((), ())),
                                preferred_element_type=jnp.float32) * scale
            mn = jnp.maximum(m, jnp.max(s, axis=1, keepdims=True))
            p = jnp.exp(s - mn)
            sc = jnp.exp(m - mn)
            l2 = l * sc + jnp.sum(p, axis=1, keepdims=True)
            acc2 = acc * sc + jnp.dot(p, v, preferred_element_type=jnp.float32)
            return mn, l2, acc2

        m0 = jnp.full((bq, 1), -1e30, jnp.float32)
        l0 = jnp.zeros((bq, 1), jnp.float32)
        a0 = jnp.zeros((bq, C), jnp.float32)
        m, l, acc = lax.fori_loop(0, n // bk, body, (m0, l0, a0))
        outs.append(acc / l)
    o = jnp.concatenate(outs, axis=1)
    agg = jnp.dot(o, ow_ref[...], preferred_element_type=jnp.float32) \
        + ob_ref[...]
    st = st_ref[...]
    gi = jnp.concatenate([st, agg], axis=1)
    hg = jnp.maximum(jnp.dot(gi, gw1_ref[...],
                             preferred_element_type=jnp.float32)
                     + gb1_ref[...], 0.0)
    s1 = jnp.sum(hg * g2_ref[...], axis=1)[:, None]
    strength = jax.nn.sigmoid(s1 + b2_ref[...])
    gated = agg * strength
    o_ref[...] = jnp.dot(gated, pw_ref[...],
                         preferred_element_type=jnp.float32) \
        + pb_ref[...] + st


def _mha_gate(qkv, states, ow, ob, gw1, gb1, g2, b2, pw, pb, bq=256, bk=512):
    n = qkv.shape[0]
    d = states.shape[1]
    full = lambda a: pl.BlockSpec(a.shape, lambda i: tuple(0 for _ in a.shape))
    kern = functools.partial(_mha_kernel, bq=bq, bk=bk, n=n)
    return pl.pallas_call(
        kern,
        grid=(n // bq,),
        in_specs=[full(qkv),
                  pl.BlockSpec((bq, qkv.shape[1]), lambda i: (i, 0)),
                  pl.BlockSpec((bq, d), lambda i: (i, 0)),
                  full(ow), full(ob), full(gw1), full(gb1),
                  full(g2), full(b2), full(pw), full(pb)],
        out_specs=pl.BlockSpec((bq, d), lambda i: (i, 0)),
        out_shape=jax.ShapeDtypeStruct((n, d), jnp.float32),
        interpret=_INTERPRET,
    )(qkv, qkv, states, ow, ob, gw1, gb1, g2, b2, pw, pb)


_EPT = 16384
_BCAP = 2048
_ARS = 8192
_CH = 128


def _cnt_sc_body(src_hbm, dst_hbm, z_hbm, cnt_hbm, acc, src_v, dst_v,
                 b0, b1, b2, b3, b4, b5, b6, b7,
                 stg, colb, idxb, stg_sl, idx_sl):
    c = lax.axis_index("c")
    t = lax.axis_index("s")
    buckets = [b0, b1, b2, b3, b4, b5, b6, b7]
    iota = lax.iota(jnp.int32, 16)

    pltpu.sync_copy(z_hbm, stg)
    pltpu.sync_copy(z_hbm.at[pl.ds(0, 16)], stg_sl)
    for u in range(_CH // 16):
        colb[pl.ds(u * 16, 16)] = jnp.zeros((16,), jnp.int32)

    _ECH = src_v.shape[0]

    def binc(e, curs):
        pltpu.sync_copy(src_hbm.at[pl.ds(t * _EPT + e * _ECH, _ECH)], src_v)
        pltpu.sync_copy(dst_hbm.at[pl.ds(t * _EPT + e * _ECH, _ECH)], dst_v)

        def binb(g, curs):
            s16 = src_v[pl.ds(g * 16, 16)]
            d16 = dst_v[pl.ds(g * 16, 16)]
            p16 = (d16 << 12) | s16
            key = d16 >> 8
            out = []
            for q in range(8):
                cur = jnp.minimum(curs[q], _BCAP)
                m = key == (2 * q + c)
                mi = m.astype(jnp.int32)
                pos = jnp.where(m, cur + plsc.cumsum(mi) - 1, _BCAP + 15)
                plsc.store_scatter(buckets[q], [pos], p16)
                out.append(cur + jnp.sum(mi))
            return tuple(out)
        return lax.fori_loop(0, _ECH // 16, binb, curs)
    nq = lax.fori_loop(0, _EPT // _ECH, binc, (jnp.int32(0),) * 8)

    rpt = _ARS // 16
    for r in range(8):
        glo = (2 * r + c) * 256
        plsc.subcore_barrier()
        for k in range(rpt // 128):
            pltpu.sync_copy(z_hbm, acc.at[pl.ds(t * rpt + k * 128, 128)])
        plsc.subcore_barrier()

        ds16 = glo + t * 16 + iota
        psl = ds16 * 4097
        slc = psl & 127
        plsc.store_scatter(stg_sl, [iota, slc], jnp.ones((16,), jnp.float32))
        idx_sl[...] = (psl >> 7) - glo * 32
        pltpu.sync_copy(stg_sl, acc.at[idx_sl], add=True)
        plsc.store_scatter(stg_sl, [iota, slc], jnp.zeros((16,), jnp.float32))

        nb = nq[r]
        nch = (nb + (_CH - 1)) // _CH

        def chunk(ch, _, r=r, glo=glo):
            for u in range(_CH // 16):
                v = ch * (_CH // 16) + u
                pv = buckets[r][pl.ds(v * 16, 16)]
                rem = nb - v * 16
                valid = iota < rem
                rows = jnp.where(valid, (pv >> 7) - glo * 32, 0)
                cols = pv & 127
                val = jnp.where(valid, 1.0, 0.0)
                prevc = colb[pl.ds(u * 16, 16)]
                plsc.store_scatter(stg, [u * 16 + iota, prevc],
                                   jnp.zeros((16,), jnp.float32))
                plsc.store_scatter(stg, [u * 16 + iota, cols], val)
                colb[pl.ds(u * 16, 16)] = cols
                idxb[pl.ds(u * 16, 16)] = rows
            pltpu.sync_copy(stg, acc.at[idxb], add=True)
            return 0
        lax.fori_loop(0, nch, chunk, 0)

        plsc.subcore_barrier()
        pltpu.sync_copy(acc.at[pl.ds(t * rpt, rpt)],
                        cnt_hbm.at[pl.ds(glo * 32 + t * rpt, rpt)])


def _build_cnt(edge_index, n):
    mesh = plsc.VectorSubcoreMesh(core_axis_name="c", subcore_axis_name="s")
    f = pl.kernel(
        _cnt_sc_body,
        mesh=mesh,
        compiler_params=pltpu.CompilerParams(needs_layout_passes=False),
        out_type=jax.ShapeDtypeStruct((n * n // 128, 128), jnp.float32),
        scratch_types=[
            pltpu.VMEM_SHARED((_ARS, 128), jnp.float32),
            pltpu.VMEM((2048,), jnp.int32),
            pltpu.VMEM((2048,), jnp.int32),
        ] + [pltpu.VMEM((_BCAP + 16,), jnp.int32) for _ in range(8)] + [
            pltpu.VMEM((_CH, 128), jnp.float32),
            pltpu.VMEM((_CH,), jnp.int32),
            pltpu.VMEM((_CH,), jnp.int32),
            pltpu.VMEM((16, 128), jnp.float32),
            pltpu.VMEM((16,), jnp.int32),
        ],
    )
    zeros = jnp.zeros((128, 128), jnp.float32)
    cnt = f(edge_index[0], edge_index[1], zeros)
    return cnt.reshape(n, n)


def _head_blockdiag(att):
    cols = [jnp.zeros((H * C,), jnp.float32).at[h * C:(h + 1) * C]
            .set(att[h]) for h in range(H)]
    return jnp.stack(cols, axis=1)


def kernel(agent_states, edge_index, params):
    p = params
    n, d = agent_states.shape

    cnt = _build_cnt(edge_index, n)

    x = agent_states + p['agent_emb'] + jnp.tile(p['role_emb'], (1, 4))
    for l in range(2):
        gp = p['gat'][l]
        s_m = _head_blockdiag(gp['att_src'])
        d_m = _head_blockdiag(gp['att_dst'])
        wc = jnp.concatenate([gp['W'], gp['W'] @ s_m, gp['W'] @ d_m], axis=1)
        t = _matmul(x, wc)
        xp = t[:, :d]
        asr = t[:, d:d + H]
        adr = t[:, d + H:d + 2 * H]
        ms = jnp.max(asr, axis=0)
        zs = adr + ms[None, :]
        b = jnp.maximum(zs, 0.2 * zs)
        a1 = jnp.exp(asr - ms[None, :])
        a2 = jnp.exp(0.2 * (asr - ms[None, :]))
        b1 = jnp.exp(zs - b)
        b2 = jnp.exp(0.2 * zs - b)
        asrc = jnp.concatenate([a1.T, a2.T], axis=0)
        bdst = jnp.concatenate([b1, b2], axis=1)
        xpr = xp.reshape(n, H, C)
        va = jnp.concatenate(
            [xpr, jnp.ones((n, H, 1), jnp.float32),
             jnp.zeros((n, H, 7), jnp.float32)], axis=2).reshape(n, 40 * H)
        x = _gat_layer(cnt, va, asrc, bdst, x, gp['bias'][None, :],
                       p['ln_g'][l][None, :], p['ln_b'][l][None, :])

    qkv = _mlp_qkv(x, p['enc_W1'], p['enc_b1'][None, :],
                   p['enc_W2'], p['enc_b2'][None, :],
                   p['dec_W1'], p['dec_b1'][None, :],
                   p['dec_W2'], p['dec_b2'][None, :],
                   p['mha_in_W'], p['mha_in_b'][None, :])

    g2 = p['gate_W2'][:, 0][None, :]
    b2 = jnp.broadcast_to(p['gate_b2'], (1, d))
    out = _mha_gate(qkv, agent_states,
                    p['mha_out_W'], p['mha_out_b'][None, :],
                    p['gate_W1'], p['gate_b1'][None, :],
                    g2, b2,
                    p['proj_W'], p['proj_b'][None, :])
    return out


# speedup vs baseline: 89.2729x; 1.2401x over previous
import functools
import math

import jax
import jax.numpy as jnp
from jax import lax
from jax.experimental import pallas as pl
from jax.experimental.pallas import tpu as pltpu
from jax.experimental.pallas import tpu_sc as plsc

_INTERPRET = False

H = 4
C = 32


def _matmul_kernel(x_ref, w_ref, o_ref):
    o_ref[...] = jnp.dot(x_ref[...], w_ref[...],
                         preferred_element_type=jnp.float32)


def _matmul(x, w, bm=512):
    n, k = x.shape
    m = w.shape[1]
    return pl.pallas_call(
        _matmul_kernel,
        grid=(n // bm,),
        in_specs=[pl.BlockSpec((bm, k), lambda i: (i, 0)),
                  pl.BlockSpec((k, m), lambda i: (0, 0))],
        out_specs=pl.BlockSpec((bm, m), lambda i: (i, 0)),
        out_shape=jax.ShapeDtypeStruct((n, m), jnp.float32),
        interpret=_INTERPRET,
    )(x, w)


def _gat_kernel(cnt_ref, va_ref, asrc_ref, bdst_ref, xres_ref, gb_ref,
                lng_ref, lnb_ref, o_ref, acc_ref, *, bd, bs, nsb):
    j = pl.program_id(1)

    @pl.when(j == 0)
    def _():
        acc_ref[...] = jnp.zeros_like(acc_ref)

    for hh in range(H):
        b2 = bdst_ref[:, hh:hh + 1]
        tot = None
        for k in range(4):
            cnt = cnt_ref[k * 256:(k + 1) * 256, :]
            a1 = asrc_ref[hh, k * 128:(k + 1) * 128][None, :]
            a2 = asrc_ref[H + hh, k * 128:(k + 1) * 128][None, :]
            w = jnp.maximum(a1, a2 * b2) * cnt
            d = jnp.dot(w, va_ref[k * 128:(k + 1) * 128,
                                  40 * hh:40 * hh + 40],
                        preferred_element_type=jnp.float32)
            tot = d if tot is None else tot + d
        acc_ref[:, 40 * hh:40 * hh + 40] += tot

    @pl.when(j == nsb - 1)
    def _():
        y = acc_ref[...]
        parts = [y[:, 40 * hh:40 * hh + C] / y[:, 40 * hh + C:40 * hh + C + 1]
                 for hh in range(H)]
        y = jnp.concatenate(parts, axis=1) + gb_ref[...]
        mu = jnp.mean(y, axis=1, keepdims=True)
        var = jnp.mean((y - mu) ** 2, axis=1, keepdims=True)
        y = (y - mu) * lax.rsqrt(var + 1e-5) * lng_ref[...] + lnb_ref[...]
        o_ref[...] = y + xres_ref[...]


def _gat_layer(cnt, va, asrc, bdst, xres, gbias, lng, lnb, bd=256, bs=512):
    n, d = xres.shape
    gd, gs = n // bd, n // bs
    kern = functools.partial(_gat_kernel, bd=bd, bs=bs, nsb=gs)
    return pl.pallas_call(
        kern,
        grid=(gd, gs),
        in_specs=[
            pl.BlockSpec((4 * bd, 128), lambda i, j: (i * 8 + j, 0)),
            pl.BlockSpec((bs, 40 * H), lambda i, j: (j, 0)),
            pl.BlockSpec((2 * H, bs), lambda i, j: (0, j)),
            pl.BlockSpec((bd, 2 * H), lambda i, j: (i, 0)),
            pl.BlockSpec((bd, d), lambda i, j: (i, 0)),
            pl.BlockSpec((1, d), lambda i, j: (0, 0)),
            pl.BlockSpec((1, d), lambda i, j: (0, 0)),
            pl.BlockSpec((1, d), lambda i, j: (0, 0)),
        ],
        out_specs=pl.BlockSpec((bd, d), lambda i, j: (i, 0)),
        out_shape=jax.ShapeDtypeStruct((n, d), jnp.float32),
        scratch_shapes=[pltpu.VMEM((bd, 40 * H), jnp.float32)],
        interpret=_INTERPRET,
    )(cnt, va, asrc, bdst, xres, gbias, lng, lnb)


def _mlp_kernel(x_ref, ew1_ref, eb1_ref, ew2_ref, eb2_ref, dw1_ref, db1_ref,
                dw2_ref, db2_ref, iw_ref, ib_ref, o_ref):
    x = x_ref[...]
    h1 = jnp.maximum(jnp.dot(x, ew1_ref[...],
                             preferred_element_type=jnp.float32)
                     + eb1_ref[...], 0.0)
    msg = jnp.dot(h1, ew2_ref[...],
                  preferred_element_type=jnp.float32) + eb2_ref[...]
    d1 = jnp.maximum(jnp.dot(msg, dw1_ref[...],
                             preferred_element_type=jnp.float32)
                     + db1_ref[...], 0.0)
    dec = jnp.dot(d1, dw2_ref[...],
                  preferred_element_type=jnp.float32) + db2_ref[...]
    o_ref[...] = jnp.dot(dec, iw_ref[...],
                         preferred_element_type=jnp.float32) + ib_ref[...]


def _mlp_qkv(x, ew1, eb1, ew2, eb2, dw1, db1, dw2, db2, iw, ib, bm=512):
    n, d = x.shape
    m = iw.shape[1]
    full = lambda a: pl.BlockSpec(a.shape, lambda i: tuple(0 for _ in a.shape))
    return pl.pallas_call(
        _mlp_kernel,
        grid=(n // bm,),
        in_specs=[pl.BlockSpec((bm, d), lambda i: (i, 0)),
                  full(ew1), full(eb1), full(ew2), full(eb2),
                  full(dw1), full(db1), full(dw2), full(db2),
                  full(iw), full(ib)],
        out_specs=pl.BlockSpec((bm, m), lambda i: (i, 0)),
        out_shape=jax.ShapeDtypeStruct((n, m), jnp.float32),
        interpret=_INTERPRET,
    )(x, ew1, eb1, ew2, eb2, dw1, db1, dw2, db2, iw, ib)


def _mha_kernel(qkv_ref, qblk_ref, st_ref, ow_ref, ob_ref, gw1_ref, gb1_ref,
                g2_ref, b2_ref, pw_ref, pb_ref, o_ref, *, bq, bk, n):
    scale = 1.0 / math.sqrt(C)
    outs = []
    for hh in range(H):
        q = qblk_ref[:, hh * C:(hh + 1) * C]

        def body(kb, carry, q=q, hh=hh):
            l, acc = carry
            k = qkv_ref[pl.ds(kb * bk, bk), 128 + hh * C:128 + (hh + 1) * C]
            v = qkv_ref[pl.ds(kb * bk, bk), 256 + hh * C:256 + (hh + 1) * C]
            s = lax.dot_general(q, k, (((1,), (1,)), ((), ())),
                                preferred_element_type=jnp.float32) * scale
            p = jnp.exp(s)
            l2 = l + jnp.sum(p, axis=1, keepdims=True)
            acc2 = acc + jnp.dot(p, v, preferred_element_type=jnp.float32)
            return l2, acc2

        l0 = jnp.zeros((bq, 1), jnp.float32)
        a0 = jnp.zeros((bq, C), jnp.float32)
        l, acc = lax.fori_loop(0, n // bk, body, (l0, a0))
        outs.append(acc / l)
    o = jnp.concatenate(outs, axis=1)
    agg = jnp.dot(o, ow_ref[...], preferred_element_type=jnp.float32) \
        + ob_ref[...]
    st = st_ref[...]
    gi = jnp.concatenate([st, agg], axis=1)
    hg = jnp.maximum(jnp.dot(gi, gw1_ref[...],
                             preferred_element_type=jnp.float32)
                     + gb1_ref[...], 0.0)
    s1 = jnp.sum(hg * g2_ref[...], axis=1)[:, None]
    strength = jax.nn.sigmoid(s1 + b2_ref[...])
    gated = agg * strength
    o_ref[...] = jnp.dot(gated, pw_ref[...],
                         preferred_element_type=jnp.float32) \
        + pb_ref[...] + st


def _mha_gate(qkv, states, ow, ob, gw1, gb1, g2, b2, pw, pb, bq=256, bk=512):
    n = qkv.shape[0]
    d = states.shape[1]
    full = lambda a: pl.BlockSpec(a.shape, lambda i: tuple(0 for _ in a.shape))
    kern = functools.partial(_mha_kernel, bq=bq, bk=bk, n=n)
    return pl.pallas_call(
        kern,
        grid=(n // bq,),
        in_specs=[full(qkv),
                  pl.BlockSpec((bq, qkv.shape[1]), lambda i: (i, 0)),
                  pl.BlockSpec((bq, d), lambda i: (i, 0)),
                  full(ow), full(ob), full(gw1), full(gb1),
                  full(g2), full(b2), full(pw), full(pb)],
        out_specs=pl.BlockSpec((bq, d), lambda i: (i, 0)),
        out_shape=jax.ShapeDtypeStruct((n, d), jnp.float32),
        interpret=_INTERPRET,
    )(qkv, qkv, states, ow, ob, gw1, gb1, g2, b2, pw, pb)


_EPT = 16384
_BCAP = 2048
_ARS = 8192
_CH = 128


def _cnt_sc_body(src_hbm, dst_hbm, z_hbm, cnt_hbm, acc, src_v, dst_v,
                 b0, b1, b2, b3, b4, b5, b6, b7,
                 stg, colb, idxb, stg_sl, idx_sl):
    c = lax.axis_index("c")
    t = lax.axis_index("s")
    buckets = [b0, b1, b2, b3, b4, b5, b6, b7]
    iota = lax.iota(jnp.int32, 16)

    pltpu.sync_copy(z_hbm, stg)
    pltpu.sync_copy(z_hbm.at[pl.ds(0, 16)], stg_sl)
    for u in range(_CH // 16):
        colb[pl.ds(u * 16, 16)] = jnp.zeros((16,), jnp.int32)

    _ECH = src_v.shape[0]

    def binc(e, curs):
        pltpu.sync_copy(src_hbm.at[pl.ds(t * _EPT + e * _ECH, _ECH)], src_v)
        pltpu.sync_copy(dst_hbm.at[pl.ds(t * _EPT + e * _ECH, _ECH)], dst_v)

        def binb(g, curs):
            s16 = src_v[pl.ds(g * 16, 16)]
            d16 = dst_v[pl.ds(g * 16, 16)]
            p16 = (d16 << 12) | s16
            key = d16 >> 8
            out = []
            for q in range(8):
                cur = jnp.minimum(curs[q], _BCAP)
                m = key == (2 * q + c)
                mi = m.astype(jnp.int32)
                pos = jnp.where(m, cur + plsc.cumsum(mi) - 1, _BCAP + 15)
                plsc.store_scatter(buckets[q], [pos], p16)
                out.append(cur + jnp.sum(mi))
            return tuple(out)
        return lax.fori_loop(0, _ECH // 16, binb, curs)
    nq = lax.fori_loop(0, _EPT // _ECH, binc, (jnp.int32(0),) * 8)

    rpt = _ARS // 16
    for r in range(8):
        glo = (2 * r + c) * 256
        plsc.subcore_barrier()
        for k in range(rpt // 128):
            pltpu.sync_copy(z_hbm, acc.at[pl.ds(t * rpt + k * 128, 128)])
        plsc.subcore_barrier()

        ds16 = glo + t * 16 + iota
        slc = ds16 & 127
        plsc.store_scatter(stg_sl, [iota, slc], jnp.ones((16,), jnp.float32))
        idx_sl[...] = (((ds16 & 4095) >> 7) << 8) | (ds16 & 255)
        pltpu.sync_copy(stg_sl, acc.at[idx_sl], add=True)
        plsc.store_scatter(stg_sl, [iota, slc], jnp.zeros((16,), jnp.float32))

        nb = nq[r]
        nch = (nb + (_CH - 1)) // _CH

        def chunk(ch, _, r=r, glo=glo):
            for u in range(_CH // 16):
                v = ch * (_CH // 16) + u
                pv = buckets[r][pl.ds(v * 16, 16)]
                rem = nb - v * 16
                valid = iota < rem
                rl = ((((pv & 4095) >> 7)) << 8) | ((pv >> 12) & 255)
                rows = jnp.where(valid, rl, 0)
                cols = pv & 127
                val = jnp.where(valid, 1.0, 0.0)
                prevc = colb[pl.ds(u * 16, 16)]
                plsc.store_scatter(stg, [u * 16 + iota, prevc],
                                   jnp.zeros((16,), jnp.float32))
                plsc.store_scatter(stg, [u * 16 + iota, cols], val)
                colb[pl.ds(u * 16, 16)] = cols
                idxb[pl.ds(u * 16, 16)] = rows
            pltpu.sync_copy(stg, acc.at[idxb], add=True)
            return 0
        lax.fori_loop(0, nch, chunk, 0)

        plsc.subcore_barrier()
        pltpu.sync_copy(acc.at[pl.ds(t * rpt, rpt)],
                        cnt_hbm.at[pl.ds((2 * r + c) * _ARS + t * rpt,
                                          rpt)])


def _build_cnt(edge_index, n):
    mesh = plsc.VectorSubcoreMesh(core_axis_name="c", subcore_axis_name="s")
    f = pl.kernel(
        _cnt_sc_body,
        mesh=mesh,
        compiler_params=pltpu.CompilerParams(needs_layout_passes=False),
        out_type=jax.ShapeDtypeStruct((n * n // 128, 128), jnp.float32),
        scratch_types=[
            pltpu.VMEM_SHARED((_ARS, 128), jnp.float32),
            pltpu.VMEM((2048,), jnp.int32),
            pltpu.VMEM((2048,), jnp.int32),
        ] + [pltpu.VMEM((_BCAP + 16,), jnp.int32) for _ in range(8)] + [
            pltpu.VMEM((_CH, 128), jnp.float32),
            pltpu.VMEM((_CH,), jnp.int32),
            pltpu.VMEM((_CH,), jnp.int32),
            pltpu.VMEM((16, 128), jnp.float32),
            pltpu.VMEM((16,), jnp.int32),
        ],
    )
    zeros = jnp.zeros((128, 128), jnp.float32)
    return f(edge_index[0], edge_index[1], zeros)


def _head_blockdiag(att):
    cols = [jnp.zeros((H * C,), jnp.float32).at[h * C:(h + 1) * C]
            .set(att[h]) for h in range(H)]
    return jnp.stack(cols, axis=1)


def kernel(agent_states, edge_index, params):
    p = params
    n, d = agent_states.shape

    cnt = _build_cnt(edge_index, n)

    x = agent_states + p['agent_emb'] + jnp.tile(p['role_emb'], (1, 4))
    for l in range(2):
        gp = p['gat'][l]
        s_m = _head_blockdiag(gp['att_src'])
        d_m = _head_blockdiag(gp['att_dst'])
        wc = jnp.concatenate([gp['W'], gp['W'] @ s_m, gp['W'] @ d_m], axis=1)
        t = _matmul(x, wc)
        xp = t[:, :d]
        asr = t[:, d:d + H]
        adr = t[:, d + H:d + 2 * H]
        ms = jnp.max(asr, axis=0)
        a1 = jnp.exp(asr - ms[None, :])
        a2 = jnp.exp(0.2 * (asr - ms[None, :]))
        b2 = jnp.exp(-0.8 * (adr + ms[None, :]))
        asrc = jnp.concatenate([a1.T, a2.T], axis=0)
        bdst = jnp.concatenate([b2, b2], axis=1)
        xpr = xp.reshape(n, H, C)
        va = jnp.concatenate(
            [xpr, jnp.ones((n, H, 1), jnp.float32),
             jnp.zeros((n, H, 7), jnp.float32)], axis=2).reshape(n, 40 * H)
        x = _gat_layer(cnt, va, asrc, bdst, x, gp['bias'][None, :],
                       p['ln_g'][l][None, :], p['ln_b'][l][None, :])

    qkv = _mlp_qkv(x, p['enc_W1'], p['enc_b1'][None, :],
                   p['enc_W2'], p['enc_b2'][None, :],
                   p['dec_W1'], p['dec_b1'][None, :],
                   p['dec_W2'], p['dec_b2'][None, :],
                   p['mha_in_W'], p['mha_in_b'][None, :])

    g2 = p['gate_W2'][:, 0][None, :]
    b2 = jnp.broadcast_to(p['gate_b2'], (1, d))
    out = _mha_gate(qkv, agent_states,
                    p['mha_out_W'], p['mha_out_b'][None, :],
                    p['gate_W1'], p['gate_b1'][None, :],
                    g2, b2,
                    p['proj_W'], p['proj_b'][None, :])
    return out


# speedup vs baseline: 95.6331x; 1.0712x over previous
import functools
import math

import jax
import jax.numpy as jnp
from jax import lax
from jax.experimental import pallas as pl
from jax.experimental.pallas import tpu as pltpu
from jax.experimental.pallas import tpu_sc as plsc

_INTERPRET = False

H = 4
C = 32


def _matmul_kernel(x_ref, w_ref, o_ref):
    o_ref[...] = jnp.dot(x_ref[...], w_ref[...],
                         preferred_element_type=jnp.float32)


def _matmul(x, w, bm=512):
    n, k = x.shape
    m = w.shape[1]
    return pl.pallas_call(
        _matmul_kernel,
        grid=(n // bm,),
        in_specs=[pl.BlockSpec((bm, k), lambda i: (i, 0)),
                  pl.BlockSpec((k, m), lambda i: (0, 0))],
        out_specs=pl.BlockSpec((bm, m), lambda i: (i, 0)),
        out_shape=jax.ShapeDtypeStruct((n, m), jnp.float32),
        interpret=_INTERPRET,
    )(x, w)


def _gat_kernel(cnt_ref, va_ref, asrc_ref, bdst_ref, xres_ref, gb_ref,
                lng_ref, lnb_ref, o_ref, acc_ref, *, bd, bs, nsb):
    j = pl.program_id(1)

    @pl.when(j == 0)
    def _():
        acc_ref[...] = jnp.zeros_like(acc_ref)

    for hh in range(H):
        b2 = bdst_ref[:, hh:hh + 1]
        tot = None
        for k in range(4):
            cnt = cnt_ref[k * 256:(k + 1) * 256, :]
            a1 = asrc_ref[hh, k * 128:(k + 1) * 128][None, :]
            a2 = asrc_ref[H + hh, k * 128:(k + 1) * 128][None, :]
            w = jnp.maximum(a1, a2 * b2) * cnt
            d = jnp.dot(w, va_ref[k * 128:(k + 1) * 128,
                                  40 * hh:40 * hh + 40],
                        preferred_element_type=jnp.float32)
            tot = d if tot is None else tot + d
        acc_ref[:, 40 * hh:40 * hh + 40] += tot

    @pl.when(j == nsb - 1)
    def _():
        y = acc_ref[...]
        parts = [y[:, 40 * hh:40 * hh + C] / y[:, 40 * hh + C:40 * hh + C + 1]
                 for hh in range(H)]
        y = jnp.concatenate(parts, axis=1) + gb_ref[...]
        mu = jnp.mean(y, axis=1, keepdims=True)
        var = jnp.mean((y - mu) ** 2, axis=1, keepdims=True)
        y = (y - mu) * lax.rsqrt(var + 1e-5) * lng_ref[...] + lnb_ref[...]
        o_ref[...] = y + xres_ref[...]


def _gat_layer(cnt, va, asrc, bdst, xres, gbias, lng, lnb, bd=256, bs=512):
    n, d = xres.shape
    gd, gs = n // bd, n // bs
    kern = functools.partial(_gat_kernel, bd=bd, bs=bs, nsb=gs)
    return pl.pallas_call(
        kern,
        grid=(gd, gs),
        in_specs=[
            pl.BlockSpec((4 * bd, 128), lambda i, j: (i * 8 + j, 0)),
            pl.BlockSpec((bs, 40 * H), lambda i, j: (j, 0)),
            pl.BlockSpec((2 * H, bs), lambda i, j: (0, j)),
            pl.BlockSpec((bd, 2 * H), lambda i, j: (i, 0)),
            pl.BlockSpec((bd, d), lambda i, j: (i, 0)),
            pl.BlockSpec((1, d), lambda i, j: (0, 0)),
            pl.BlockSpec((1, d), lambda i, j: (0, 0)),
            pl.BlockSpec((1, d), lambda i, j: (0, 0)),
        ],
        out_specs=pl.BlockSpec((bd, d), lambda i, j: (i, 0)),
        out_shape=jax.ShapeDtypeStruct((n, d), jnp.float32),
        scratch_shapes=[pltpu.VMEM((bd, 40 * H), jnp.float32)],
        interpret=_INTERPRET,
    )(cnt, va, asrc, bdst, xres, gbias, lng, lnb)


def _mlp_kernel(x_ref, ew1_ref, eb1_ref, ew2_ref, eb2_ref, dw1_ref, db1_ref,
                dw2_ref, db2_ref, iw_ref, ib_ref, o_ref):
    x = x_ref[...]
    h1 = jnp.maximum(jnp.dot(x, ew1_ref[...],
                             preferred_element_type=jnp.float32)
                     + eb1_ref[...], 0.0)
    msg = jnp.dot(h1, ew2_ref[...],
                  preferred_element_type=jnp.float32) + eb2_ref[...]
    d1 = jnp.maximum(jnp.dot(msg, dw1_ref[...],
                             preferred_element_type=jnp.float32)
                     + db1_ref[...], 0.0)
    dec = jnp.dot(d1, dw2_ref[...],
                  preferred_element_type=jnp.float32) + db2_ref[...]
    o_ref[...] = jnp.dot(dec, iw_ref[...],
                         preferred_element_type=jnp.float32) + ib_ref[...]


def _mlp_qkv(x, ew1, eb1, ew2, eb2, dw1, db1, dw2, db2, iw, ib, bm=512):
    n, d = x.shape
    m = iw.shape[1]
    full = lambda a: pl.BlockSpec(a.shape, lambda i: tuple(0 for _ in a.shape))
    return pl.pallas_call(
        _mlp_kernel,
        grid=(n // bm,),
        in_specs=[pl.BlockSpec((bm, d), lambda i: (i, 0)),
                  full(ew1), full(eb1), full(ew2), full(eb2),
                  full(dw1), full(db1), full(dw2), full(db2),
                  full(iw), full(ib)],
        out_specs=pl.BlockSpec((bm, m), lambda i: (i, 0)),
        out_shape=jax.ShapeDtypeStruct((n, m), jnp.float32),
        interpret=_INTERPRET,
    )(x, ew1, eb1, ew2, eb2, dw1, db1, dw2, db2, iw, ib)


def _mha_kernel(qkv_ref, qblk_ref, st_ref, ow_ref, ob_ref, gw1_ref, gb1_ref,
                g2_ref, b2_ref, pw_ref, pb_ref, o_ref, *, bq, bk, n):
    outs = []
    for hh in range(H):
        q = qblk_ref[:, hh * C:(hh + 1) * C]

        def body(kb, acc, q=q, hh=hh):
            k = qkv_ref[pl.ds(kb * bk, bk), 128 + hh * C:128 + (hh + 1) * C]
            va = qkv_ref[pl.ds(kb * bk, bk), 256 + 40 * hh:256 + 40 * hh + 40]
            s = lax.dot_general(q, k, (((1,), (1,)), ((), ())),
                                preferred_element_type=jnp.float32)
            p = jnp.exp(s)
            return acc + jnp.dot(p, va, preferred_element_type=jnp.float32)

        a0 = jnp.zeros((bq, 40), jnp.float32)
        acc = lax.fori_loop(0, n // bk, body, a0)
        outs.append(acc[:, :C] / acc[:, C:C + 1])
    o = jnp.concatenate(outs, axis=1)
    agg = jnp.dot(o, ow_ref[...], preferred_element_type=jnp.float32) \
        + ob_ref[...]
    st = st_ref[...]
    gi = jnp.concatenate([st, agg], axis=1)
    hg = jnp.maximum(jnp.dot(gi, gw1_ref[...],
                             preferred_element_type=jnp.float32)
                     + gb1_ref[...], 0.0)
    s1 = jnp.sum(hg * g2_ref[...], axis=1)[:, None]
    strength = jax.nn.sigmoid(s1 + b2_ref[...])
    gated = agg * strength
    o_ref[...] = jnp.dot(gated, pw_ref[...],
                         preferred_element_type=jnp.float32) \
        + pb_ref[...] + st


def _mha_gate(qkv, states, ow, ob, gw1, gb1, g2, b2, pw, pb, bq=256, bk=512):
    n = qkv.shape[0]
    d = states.shape[1]
    full = lambda a: pl.BlockSpec(a.shape, lambda i: tuple(0 for _ in a.shape))
    kern = functools.partial(_mha_kernel, bq=bq, bk=bk, n=n)
    return pl.pallas_call(
        kern,
        grid=(n // bq,),
        in_specs=[full(qkv),
                  pl.BlockSpec((bq, qkv.shape[1]), lambda i: (i, 0)),
                  pl.BlockSpec((bq, d), lambda i: (i, 0)),
                  full(ow), full(ob), full(gw1), full(gb1),
                  full(g2), full(b2), full(pw), full(pb)],
        out_specs=pl.BlockSpec((bq, d), lambda i: (i, 0)),
        out_shape=jax.ShapeDtypeStruct((n, d), jnp.float32),
        interpret=_INTERPRET,
    )(qkv, qkv, states, ow, ob, gw1, gb1, g2, b2, pw, pb)


_EPT = 16384
_BCAP = 2048
_ARS = 8192
_CH = 128


def _cnt_sc_body(src_hbm, dst_hbm, z_hbm, cnt_hbm, acc, src_v, dst_v,
                 b0, b1, b2, b3, b4, b5, b6, b7,
                 stg, colb, idxb, stg_sl, idx_sl, ztile):
    c = lax.axis_index("c")
    t = lax.axis_index("s")
    buckets = [b0, b1, b2, b3, b4, b5, b6, b7]
    iota = lax.iota(jnp.int32, 16)

    pltpu.sync_copy(z_hbm, stg)
    pltpu.sync_copy(z_hbm, ztile)
    pltpu.sync_copy(z_hbm.at[pl.ds(0, 16)], stg_sl)
    for u in range(_CH // 16):
        colb[pl.ds(u * 16, 16)] = jnp.zeros((16,), jnp.int32)

    _ECH = src_v.shape[0]

    def binc(e, curs):
        pltpu.sync_copy(src_hbm.at[pl.ds(t * _EPT + e * _ECH, _ECH)], src_v)
        pltpu.sync_copy(dst_hbm.at[pl.ds(t * _EPT + e * _ECH, _ECH)], dst_v)

        def binb(g, curs):
            s16 = src_v[pl.ds(g * 16, 16)]
            d16 = dst_v[pl.ds(g * 16, 16)]
            p16 = (d16 << 12) | s16
            key = d16 >> 8
            out = []
            for q in range(8):
                cur = jnp.minimum(curs[q], _BCAP)
                m = key == (2 * q + c)
                mi = m.astype(jnp.int32)
                pos = jnp.where(m, cur + plsc.cumsum(mi) - 1, _BCAP + 15)
                plsc.store_scatter(buckets[q], [pos], p16)
                out.append(cur + jnp.sum(mi))
            return tuple(out)
        return lax.fori_loop(0, _ECH // 16, binb, curs)
    nq = lax.fori_loop(0, _EPT // _ECH, binc, (jnp.int32(0),) * 8)

    rpt = _ARS // 16
    for r in range(8):
        glo = (2 * r + c) * 256
        plsc.subcore_barrier()
        for k in range(rpt // 128):
            pltpu.sync_copy(ztile, acc.at[pl.ds(t * rpt + k * 128, 128)])
        plsc.subcore_barrier()

        ds16 = glo + t * 16 + iota
        slc = ds16 & 127
        plsc.store_scatter(stg_sl, [iota, slc], jnp.ones((16,), jnp.float32))
        idx_sl[...] = (((ds16 & 4095) >> 7) << 8) | (ds16 & 255)
        pltpu.sync_copy(stg_sl, acc.at[idx_sl], add=True)
        plsc.store_scatter(stg_sl, [iota, slc], jnp.zeros((16,), jnp.float32))

        nb = nq[r]
        nch = (nb + (_CH - 1)) // _CH

        def chunk(ch, _, r=r, glo=glo):
            for u in range(_CH // 16):
                v = ch * (_CH // 16) + u
                pv = buckets[r][pl.ds(v * 16, 16)]
                rem = nb - v * 16
                valid = iota < rem
                rl = ((((pv & 4095) >> 7)) << 8) | ((pv >> 12) & 255)
                rows = jnp.where(valid, rl, 0)
                cols = pv & 127
                val = jnp.where(valid, 1.0, 0.0)
                prevc = colb[pl.ds(u * 16, 16)]
                plsc.store_scatter(stg, [u * 16 + iota, prevc],
                                   jnp.zeros((16,), jnp.float32))
                plsc.store_scatter(stg, [u * 16 + iota, cols], val)
                colb[pl.ds(u * 16, 16)] = cols
                idxb[pl.ds(u * 16, 16)] = rows
            pltpu.sync_copy(stg, acc.at[idxb], add=True)
            return 0
        lax.fori_loop(0, nch, chunk, 0)

        plsc.subcore_barrier()
        pltpu.sync_copy(acc.at[pl.ds(t * rpt, rpt)],
                        cnt_hbm.at[pl.ds((2 * r + c) * _ARS + t * rpt,
                                          rpt)])


def _build_cnt(edge_index, n):
    mesh = plsc.VectorSubcoreMesh(core_axis_name="c", subcore_axis_name="s")
    f = pl.kernel(
        _cnt_sc_body,
        mesh=mesh,
        compiler_params=pltpu.CompilerParams(needs_layout_passes=False),
        out_type=jax.ShapeDtypeStruct((n * n // 128, 128), jnp.float32),
        scratch_types=[
            pltpu.VMEM_SHARED((_ARS, 128), jnp.float32),
            pltpu.VMEM((2048,), jnp.int32),
            pltpu.VMEM((2048,), jnp.int32),
        ] + [pltpu.VMEM((_BCAP + 16,), jnp.int32) for _ in range(8)] + [
            pltpu.VMEM((_CH, 128), jnp.float32),
            pltpu.VMEM((_CH,), jnp.int32),
            pltpu.VMEM((_CH,), jnp.int32),
            pltpu.VMEM((16, 128), jnp.float32),
            pltpu.VMEM((16,), jnp.int32),
            pltpu.VMEM((128, 128), jnp.float32),
        ],
    )
    zeros = jnp.zeros((128, 128), jnp.float32)
    return f(edge_index[0], edge_index[1], zeros)


def _head_blockdiag(att):
    cols = [jnp.zeros((H * C,), jnp.float32).at[h * C:(h + 1) * C]
            .set(att[h]) for h in range(H)]
    return jnp.stack(cols, axis=1)


def kernel(agent_states, edge_index, params):
    p = params
    n, d = agent_states.shape

    cnt = _build_cnt(edge_index, n)

    x = agent_states + p['agent_emb'] + jnp.tile(p['role_emb'], (1, 4))
    for l in range(2):
        gp = p['gat'][l]
        s_m = _head_blockdiag(gp['att_src'])
        d_m = _head_blockdiag(gp['att_dst'])
        wc = jnp.concatenate([gp['W'], gp['W'] @ s_m, gp['W'] @ d_m], axis=1)
        t = _matmul(x, wc)
        xp = t[:, :d]
        asr = t[:, d:d + H]
        adr = t[:, d + H:d + 2 * H]
        ms = jnp.max(asr, axis=0)
        a1 = jnp.exp(asr - ms[None, :])
        a2 = jnp.exp(0.2 * (asr - ms[None, :]))
        b2 = jnp.exp(-0.8 * (adr + ms[None, :]))
        asrc = jnp.concatenate([a1.T, a2.T], axis=0)
        bdst = jnp.concatenate([b2, b2], axis=1)
        xpr = xp.reshape(n, H, C)
        va = jnp.concatenate(
            [xpr, jnp.ones((n, H, 1), jnp.float32),
             jnp.zeros((n, H, 7), jnp.float32)], axis=2).reshape(n, 40 * H)
        x = _gat_layer(cnt, va, asrc, bdst, x, gp['bias'][None, :],
                       p['ln_g'][l][None, :], p['ln_b'][l][None, :])

    scale = 1.0 / math.sqrt(C)
    iw = p['mha_in_W']
    ib = p['mha_in_b']
    iw_parts = [iw[:, :d] * scale, iw[:, d:2 * d]]
    ib_parts = [ib[:d] * scale, ib[d:2 * d]]
    zc = jnp.zeros((d, 8), jnp.float32)
    oc = jnp.concatenate([jnp.ones((1,), jnp.float32),
                          jnp.zeros((7,), jnp.float32)])
    for hh in range(H):
        iw_parts += [iw[:, 2 * d + C * hh:2 * d + C * (hh + 1)], zc]
        ib_parts += [ib[2 * d + C * hh:2 * d + C * (hh + 1)], oc]
    iw2 = jnp.concatenate(iw_parts, axis=1)
    ib2 = jnp.concatenate(ib_parts)
    qkv = _mlp_qkv(x, p['enc_W1'], p['enc_b1'][None, :],
                   p['enc_W2'], p['enc_b2'][None, :],
                   p['dec_W1'], p['dec_b1'][None, :],
                   p['dec_W2'], p['dec_b2'][None, :],
                   iw2, ib2[None, :])

    g2 = p['gate_W2'][:, 0][None, :]
    b2 = jnp.broadcast_to(p['gate_b2'], (1, d))
    out = _mha_gate(qkv, agent_states,
                    p['mha_out_W'], p['mha_out_b'][None, :],
                    p['gate_W1'], p['gate_b1'][None, :],
                    g2, b2,
                    p['proj_W'], p['proj_b'][None, :])
    return out


# speedup vs baseline: 98.7530x; 1.0326x over previous
import functools
import math

import jax
import jax.numpy as jnp
from jax import lax
from jax.experimental import pallas as pl
from jax.experimental.pallas import tpu as pltpu
from jax.experimental.pallas import tpu_sc as plsc

_INTERPRET = False

H = 4
C = 32


def _matmul_kernel(x_ref, w_ref, o_ref):
    o_ref[...] = jnp.dot(x_ref[...], w_ref[...],
                         preferred_element_type=jnp.float32)


def _matmul(x, w, bm=512):
    n, k = x.shape
    m = w.shape[1]
    return pl.pallas_call(
        _matmul_kernel,
        grid=(n // bm,),
        in_specs=[pl.BlockSpec((bm, k), lambda i: (i, 0)),
                  pl.BlockSpec((k, m), lambda i: (0, 0))],
        out_specs=pl.BlockSpec((bm, m), lambda i: (i, 0)),
        out_shape=jax.ShapeDtypeStruct((n, m), jnp.float32),
        interpret=_INTERPRET,
    )(x, w)


def _gat_kernel(cnt_ref, va_ref, asrc_ref, bdst_ref, xres_ref, gb_ref,
                lng_ref, lnb_ref, o_ref, acc_ref, *, bd, bs, nsb):
    j = pl.program_id(1)

    @pl.when(j == 0)
    def _():
        acc_ref[...] = jnp.zeros_like(acc_ref)

    for hh in range(H):
        b2 = bdst_ref[:, hh:hh + 1]
        tot = None
        for k in range(4):
            cnt = cnt_ref[k * 256:(k + 1) * 256, :]
            a1 = asrc_ref[hh, k * 128:(k + 1) * 128][None, :]
            a2 = asrc_ref[H + hh, k * 128:(k + 1) * 128][None, :]
            w = jnp.maximum(a1, a2 * b2) * cnt
            d = jnp.dot(w, va_ref[k * 128:(k + 1) * 128,
                                  40 * hh:40 * hh + 40],
                        preferred_element_type=jnp.float32)
            tot = d if tot is None else tot + d
        acc_ref[:, 40 * hh:40 * hh + 40] += tot

    @pl.when(j == nsb - 1)
    def _():
        y = acc_ref[...]
        parts = [y[:, 40 * hh:40 * hh + C] / y[:, 40 * hh + C:40 * hh + C + 1]
                 for hh in range(H)]
        y = jnp.concatenate(parts, axis=1) + gb_ref[...]
        mu = jnp.mean(y, axis=1, keepdims=True)
        var = jnp.mean((y - mu) ** 2, axis=1, keepdims=True)
        y = (y - mu) * lax.rsqrt(var + 1e-5) * lng_ref[...] + lnb_ref[...]
        o_ref[...] = y + xres_ref[...]


def _gat_layer(cnt, va, asrc, bdst, xres, gbias, lng, lnb, bd=256, bs=512):
    n, d = xres.shape
    gd, gs = n // bd, va.shape[0] // bs
    kern = functools.partial(_gat_kernel, bd=bd, bs=bs, nsb=gs)
    return pl.pallas_call(
        kern,
        grid=(gd, gs),
        in_specs=[
            pl.BlockSpec((4 * bd, 128), lambda i, j: (i * 8 + j, 0)),
            pl.BlockSpec((bs, 40 * H), lambda i, j: (j, 0)),
            pl.BlockSpec((2 * H, bs), lambda i, j: (0, j)),
            pl.BlockSpec((bd, 2 * H), lambda i, j: (i, 0)),
            pl.BlockSpec((bd, d), lambda i, j: (i, 0)),
            pl.BlockSpec((1, d), lambda i, j: (0, 0)),
            pl.BlockSpec((1, d), lambda i, j: (0, 0)),
            pl.BlockSpec((1, d), lambda i, j: (0, 0)),
        ],
        out_specs=pl.BlockSpec((bd, d), lambda i, j: (i, 0)),
        out_shape=jax.ShapeDtypeStruct((n, d), jnp.float32),
        scratch_shapes=[pltpu.VMEM((bd, 40 * H), jnp.float32)],
        interpret=_INTERPRET,
    )(cnt, va, asrc, bdst, xres, gbias, lng, lnb)


def _mlp_kernel(x_ref, ew1_ref, eb1_ref, ew2_ref, eb2_ref, dw1_ref, db1_ref,
                dw2_ref, db2_ref, iw_ref, ib_ref, o_ref):
    x = x_ref[...]
    h1 = jnp.maximum(jnp.dot(x, ew1_ref[...],
                             preferred_element_type=jnp.float32)
                     + eb1_ref[...], 0.0)
    msg = jnp.dot(h1, ew2_ref[...],
                  preferred_element_type=jnp.float32) + eb2_ref[...]
    d1 = jnp.maximum(jnp.dot(msg, dw1_ref[...],
                             preferred_element_type=jnp.float32)
                     + db1_ref[...], 0.0)
    dec = jnp.dot(d1, dw2_ref[...],
                  preferred_element_type=jnp.float32) + db2_ref[...]
    o_ref[...] = jnp.dot(dec, iw_ref[...],
                         preferred_element_type=jnp.float32) + ib_ref[...]


def _mlp_qkv(x, ew1, eb1, ew2, eb2, dw1, db1, dw2, db2, iw, ib, bm=512):
    n, d = x.shape
    m = iw.shape[1]
    full = lambda a: pl.BlockSpec(a.shape, lambda i: tuple(0 for _ in a.shape))
    return pl.pallas_call(
        _mlp_kernel,
        grid=(n // bm,),
        in_specs=[pl.BlockSpec((bm, d), lambda i: (i, 0)),
                  full(ew1), full(eb1), full(ew2), full(eb2),
                  full(dw1), full(db1), full(dw2), full(db2),
                  full(iw), full(ib)],
        out_specs=pl.BlockSpec((bm, m), lambda i: (i, 0)),
        out_shape=jax.ShapeDtypeStruct((n, m), jnp.float32),
        interpret=_INTERPRET,
    )(x, ew1, eb1, ew2, eb2, dw1, db1, dw2, db2, iw, ib)


def _mha_kernel(qkv_ref, qblk_ref, st_ref, ow_ref, ob_ref, gw1_ref, gb1_ref,
                g2_ref, b2_ref, pw_ref, pb_ref, o_ref, *, bq, bk, n):
    outs = []
    for hh in range(H):
        q = qblk_ref[:, hh * C:(hh + 1) * C]

        def body(kb, acc, q=q, hh=hh):
            k = qkv_ref[pl.ds(kb * bk, bk), 128 + hh * C:128 + (hh + 1) * C]
            va = qkv_ref[pl.ds(kb * bk, bk), 256 + 40 * hh:256 + 40 * hh + 40]
            s = lax.dot_general(q, k, (((1,), (1,)), ((), ())),
                                preferred_element_type=jnp.float32)
            p = jnp.exp(s)
            return acc + jnp.dot(p, va, preferred_element_type=jnp.float32)

        a0 = jnp.zeros((bq, 40), jnp.float32)
        acc = lax.fori_loop(0, n // bk, body, a0)
        outs.append(acc[:, :C] / acc[:, C:C + 1])
    o = jnp.concatenate(outs, axis=1)
    agg = jnp.dot(o, ow_ref[...], preferred_element_type=jnp.float32) \
        + ob_ref[...]
    st = st_ref[...]
    gi = jnp.concatenate([st, agg], axis=1)
    hg = jnp.maximum(jnp.dot(gi, gw1_ref[...],
                             preferred_element_type=jnp.float32)
                     + gb1_ref[...], 0.0)
    s1 = jnp.sum(hg * g2_ref[...], axis=1)[:, None]
    strength = jax.nn.sigmoid(s1 + b2_ref[...])
    gated = agg * strength
    o_ref[...] = jnp.dot(gated, pw_ref[...],
                         preferred_element_type=jnp.float32) \
        + pb_ref[...] + st


def _mha_gate(qkv, states, ow, ob, gw1, gb1, g2, b2, pw, pb, bq=256, bk=512):
    n = qkv.shape[0]
    d = states.shape[1]
    full = lambda a: pl.BlockSpec(a.shape, lambda i: tuple(0 for _ in a.shape))
    kern = functools.partial(_mha_kernel, bq=bq, bk=bk, n=n)
    return pl.pallas_call(
        kern,
        grid=(n // bq,),
        in_specs=[full(qkv),
                  pl.BlockSpec((bq, qkv.shape[1]), lambda i: (i, 0)),
                  pl.BlockSpec((bq, d), lambda i: (i, 0)),
                  full(ow), full(ob), full(gw1), full(gb1),
                  full(g2), full(b2), full(pw), full(pb)],
        out_specs=pl.BlockSpec((bq, d), lambda i: (i, 0)),
        out_shape=jax.ShapeDtypeStruct((n, d), jnp.float32),
        interpret=_INTERPRET,
    )(qkv, qkv, states, ow, ob, gw1, gb1, g2, b2, pw, pb)


_EPT = 16384
_BCAP = 2048
_ARS = 8192
_CH = 128


def _cnt_sc_body(src_hbm, dst_hbm, z_hbm, cnt_hbm, acc, src_v, dst_v,
                 b0, b1, b2, b3,
                 stg, colb, idxb, stg_sl, idx_sl, ztile, *, r0):
    c = lax.axis_index("c")
    t = lax.axis_index("s")
    buckets = [b0, b1, b2, b3]
    iota = lax.iota(jnp.int32, 16)

    pltpu.sync_copy(z_hbm, stg)
    pltpu.sync_copy(z_hbm, ztile)
    pltpu.sync_copy(z_hbm.at[pl.ds(0, 16)], stg_sl)
    for u in range(_CH // 16):
        colb[pl.ds(u * 16, 16)] = jnp.zeros((16,), jnp.int32)

    _ECH = src_v.shape[0]

    def binc(e, curs):
        pltpu.sync_copy(src_hbm.at[pl.ds(t * _EPT + e * _ECH, _ECH)], src_v)
        pltpu.sync_copy(dst_hbm.at[pl.ds(t * _EPT + e * _ECH, _ECH)], dst_v)

        def binb(g, curs):
            s16 = src_v[pl.ds(g * 16, 16)]
            d16 = dst_v[pl.ds(g * 16, 16)]
            p16 = (d16 << 12) | s16
            key = d16 >> 8
            out = []
            for q in range(4):
                cur = jnp.minimum(curs[q], _BCAP)
                m = key == (2 * (r0 + q) + c)
                mi = m.astype(jnp.int32)
                pos = jnp.where(m, cur + plsc.cumsum(mi) - 1, _BCAP + 15)
                plsc.store_scatter(buckets[q], [pos], p16)
                out.append(cur + jnp.sum(mi))
            return tuple(out)
        return lax.fori_loop(0, _ECH // 16, binb, curs)
    nq = lax.fori_loop(0, _EPT // _ECH, binc, (jnp.int32(0),) * 4)

    rpt = _ARS // 16
    for r in range(r0, r0 + 4):
        glo = (2 * r + c) * 256
        plsc.subcore_barrier()
        for k in range(rpt // 128):
            pltpu.sync_copy(ztile, acc.at[pl.ds(t * rpt + k * 128, 128)])
        plsc.subcore_barrier()

        ds16 = glo + t * 16 + iota
        slc = ds16 & 127
        plsc.store_scatter(stg_sl, [iota, slc], jnp.ones((16,), jnp.float32))
        idx_sl[...] = (((ds16 & 4095) >> 7) << 8) | (ds16 & 255)
        pltpu.sync_copy(stg_sl, acc.at[idx_sl], add=True)
        plsc.store_scatter(stg_sl, [iota, slc], jnp.zeros((16,), jnp.float32))

        nb = nq[r - r0]
        nch = (nb + (_CH - 1)) // _CH

        def chunk(ch, _, r=r, glo=glo):
            for u in range(_CH // 16):
                v = ch * (_CH // 16) + u
                pv = buckets[r - r0][pl.ds(v * 16, 16)]
                rem = nb - v * 16
                valid = iota < rem
                rl = ((((pv & 4095) >> 7)) << 8) | ((pv >> 12) & 255)
                rows = jnp.where(valid, rl, 0)
                cols = pv & 127
                val = jnp.where(valid, 1.0, 0.0)
                prevc = colb[pl.ds(u * 16, 16)]
                plsc.store_scatter(stg, [u * 16 + iota, prevc],
                                   jnp.zeros((16,), jnp.float32))
                plsc.store_scatter(stg, [u * 16 + iota, cols], val)
                colb[pl.ds(u * 16, 16)] = cols
                idxb[pl.ds(u * 16, 16)] = rows
            pltpu.sync_copy(stg, acc.at[idxb], add=True)
            return 0
        lax.fori_loop(0, nch, chunk, 0)

        plsc.subcore_barrier()
        pltpu.sync_copy(acc.at[pl.ds(t * rpt, rpt)],
                        cnt_hbm.at[pl.ds((2 * (r - r0) + c) * _ARS + t * rpt,
                                          rpt)])


def _build_cnt(edge_index, n):
    mesh = plsc.VectorSubcoreMesh(core_axis_name="c", subcore_axis_name="s")
    zeros = jnp.zeros((128, 128), jnp.float32)
    halves = []
    for r0 in (0, 4):
        f = pl.kernel(
            functools.partial(_cnt_sc_body, r0=r0),
            mesh=mesh,
            compiler_params=pltpu.CompilerParams(needs_layout_passes=False),
            out_type=jax.ShapeDtypeStruct((n * n // 256, 128), jnp.float32),
            scratch_types=[
                pltpu.VMEM_SHARED((_ARS, 128), jnp.float32),
                pltpu.VMEM((2048,), jnp.int32),
                pltpu.VMEM((2048,), jnp.int32),
            ] + [pltpu.VMEM((_BCAP + 16,), jnp.int32) for _ in range(4)] + [
                pltpu.VMEM((_CH, 128), jnp.float32),
                pltpu.VMEM((_CH,), jnp.int32),
                pltpu.VMEM((_CH,), jnp.int32),
                pltpu.VMEM((16, 128), jnp.float32),
                pltpu.VMEM((16,), jnp.int32),
                pltpu.VMEM((128, 128), jnp.float32),
            ],
        )
        halves.append(f(edge_index[0], edge_index[1], zeros))
    return halves


def _head_blockdiag(att):
    cols = [jnp.zeros((H * C,), jnp.float32).at[h * C:(h + 1) * C]
            .set(att[h]) for h in range(H)]
    return jnp.stack(cols, axis=1)


def kernel(agent_states, edge_index, params):
    p = params
    n, d = agent_states.shape

    cnt = _build_cnt(edge_index, n)

    x = agent_states + p['agent_emb'] + jnp.tile(p['role_emb'], (1, 4))
    for l in range(2):
        gp = p['gat'][l]
        s_m = _head_blockdiag(gp['att_src'])
        d_m = _head_blockdiag(gp['att_dst'])
        wc = jnp.concatenate([gp['W'], gp['W'] @ s_m, gp['W'] @ d_m], axis=1)
        t = _matmul(x, wc)
        xp = t[:, :d]
        asr = t[:, d:d + H]
        adr = t[:, d + H:d + 2 * H]
        ms = jnp.max(asr, axis=0)
        a1 = jnp.exp(asr - ms[None, :])
        a2 = jnp.exp(0.2 * (asr - ms[None, :]))
        b2 = jnp.exp(-0.8 * (adr + ms[None, :]))
        asrc = jnp.concatenate([a1.T, a2.T], axis=0)
        bdst = jnp.concatenate([b2, b2], axis=1)
        xpr = xp.reshape(n, H, C)
        va = jnp.concatenate(
            [xpr, jnp.ones((n, H, 1), jnp.float32),
             jnp.zeros((n, H, 7), jnp.float32)], axis=2).reshape(n, 40 * H)
        hn = n // 2
        xt = _gat_layer(cnt[0], va, asrc, bdst[:hn], x[:hn],
                        gp['bias'][None, :], p['ln_g'][l][None, :],
                        p['ln_b'][l][None, :])
        xb = _gat_layer(cnt[1], va, asrc, bdst[hn:], x[hn:],
                        gp['bias'][None, :], p['ln_g'][l][None, :],
                        p['ln_b'][l][None, :])
        x = jnp.concatenate([xt, xb], axis=0)

    scale = 1.0 / math.sqrt(C)
    iw = p['mha_in_W']
    ib = p['mha_in_b']
    iw_parts = [iw[:, :d] * scale, iw[:, d:2 * d]]
    ib_parts = [ib[:d] * scale, ib[d:2 * d]]
    zc = jnp.zeros((d, 8), jnp.float32)
    oc = jnp.concatenate([jnp.ones((1,), jnp.float32),
                          jnp.zeros((7,), jnp.float32)])
    for hh in range(H):
        iw_parts += [iw[:, 2 * d + C * hh:2 * d + C * (hh + 1)], zc]
        ib_parts += [ib[2 * d + C * hh:2 * d + C * (hh + 1)], oc]
    iw2 = jnp.concatenate(iw_parts, axis=1)
    ib2 = jnp.concatenate(ib_parts)
    qkv = _mlp_qkv(x, p['enc_W1'], p['enc_b1'][None, :],
                   p['enc_W2'], p['enc_b2'][None, :],
                   p['dec_W1'], p['dec_b1'][None, :],
                   p['dec_W2'], p['dec_b2'][None, :],
                   iw2, ib2[None, :])

    g2 = p['gate_W2'][:, 0][None, :]
    b2 = jnp.broadcast_to(p['gate_b2'], (1, d))
    out = _mha_gate(qkv, agent_states,
                    p['mha_out_W'], p['mha_out_b'][None, :],
                    p['gate_W1'], p['gate_b1'][None, :],
                    g2, b2,
                    p['proj_W'], p['proj_b'][None, :])
    return out
